# baseline (speedup 1.0000x reference)
import functools
import math

import jax
import jax.numpy as jnp
from jax import lax
from jax.experimental import pallas as pl
from jax.experimental.pallas import tpu as pltpu
from jax.experimental.pallas import tpu_sc as plsc

EMB = 64
LANES = 16
GRP = 128
SCALE = math.sqrt(EMB) / 3.0


@functools.partial(jax.jit, static_argnames=("num_cores", "num_subcores", "chunk"))
def _three_hot_sc(tok2, emb_i, emb_v, emb_f,
                  num_cores=2, num_subcores=16, chunk=256):
    B = tok2.shape[0]
    NW = num_cores * num_subcores
    per_w = B // NW
    groups = chunk // GRP
    n_chunks = per_w // chunk
    assert n_chunks % 2 == 0 and n_chunks >= 6

    mesh = plsc.VectorSubcoreMesh(core_axis_name="c", subcore_axis_name="s")

    raw_t = pltpu.VMEM((chunk, 3), jnp.int32)
    idx_t = pltpu.VMEM((groups, GRP), jnp.int32)
    buf_t = pltpu.VMEM((chunk, EMB), jnp.float32)

    @functools.partial(
        pl.kernel,
        out_type=jax.ShapeDtypeStruct((B, EMB), jnp.float32),
        mesh=mesh,
        compiler_params=pltpu.CompilerParams(
            use_tc_tiling_on_sc=False, needs_layout_passes=False),
        scratch_types=[
            [raw_t] * 2,
            [idx_t] * 3, [idx_t] * 3,
            [buf_t] * 3, [buf_t] * 3,
            [pltpu.SemaphoreType.DMA] * 2,
            [pltpu.SemaphoreType.DMA] * 2,
            pltpu.SemaphoreType.DMA,
        ],
    )
    def kern(tok, ti, tv, tf, out, raw, x0, x1, b0, b1, gsem, osem, isem):
        wid = lax.axis_index("s") * num_cores + lax.axis_index("c")
        xs = (x0, x1)
        bs = (b0, b1)
        tabs = (ti, tv, tf)
        lane = lax.iota(jnp.int32, LANES)

        def tok0_of(c):
            return wid * per_w + c * chunk

        def fire_idx(c, p):
            pltpu.async_copy(tok.at[pl.ds(tok0_of(c), chunk)], raw[p], isem)

        def wait_idx(p):
            pltpu.make_async_copy(
                tok.at[pl.ds(0, chunk)], raw[p], isem).wait()

        def deinterleave(p):
            for f in range(3):
                col = jnp.full((LANES,), f, jnp.int32)
                for j in range(groups):
                    for k in range(GRP // LANES):
                        row = lane + (j * GRP + k * LANES)
                        xs[p][f][j, pl.ds(k * LANES, LANES)] = (
                            plsc.load_gather(raw[p], [row, col]))

        def fire_gathers(p):
            for t in range(3):
                for j in range(groups):
                    pltpu.async_copy(
                        tabs[t].at[xs[p][t].at[j]],
                        bs[p][t].at[pl.ds(j * GRP, GRP)], gsem[p])

        def drain_gathers(p):
            for t in range(3):
                for j in range(groups):
                    pltpu.make_async_copy(
                        tabs[t].at[pl.ds(0, GRP)],
                        bs[p][t].at[pl.ds(j * GRP, GRP)], gsem[p]).wait()

        def compute(p):
            bi, bv, bf = bs[p]

            def row_body(r, _):
                for q in range(EMB // LANES):
                    s = pl.ds(q * LANES, LANES)
                    bi[r, s] = (bi[r, s] + bv[r, s] + bf[r, s]) * SCALE
                return ()

            lax.fori_loop(0, chunk, row_body, ())

        def fire_out(c, p):
            pltpu.async_copy(
                bs[p][0], out.at[pl.ds(tok0_of(c), chunk)], osem[p])

        def drain_out(p):
            pltpu.make_async_copy(
                bs[p][0], out.at[pl.ds(0, chunk)], osem[p]).wait()

        def iteration(c, p, *, first=False, fire_next=True, fire_idx2=True):
            q = p ^ 1
            if not first:
                drain_out(q)
            if fire_next:
                wait_idx(q)
                deinterleave(q)
                fire_gathers(q)
            drain_gathers(p)
            if fire_idx2:
                fire_idx(c + 2, p)
            compute(p)
            fire_out(c, p)

        pltpu.sync_copy(tok.at[pl.ds(tok0_of(0), chunk)], raw[0])
        deinterleave(0)
        fire_gathers(0)
        fire_idx(1, 1)

        iteration(0, 0, first=True)
        iteration(1, 1)

        @pl.loop(2, n_chunks - 2, step=2)
        def steady(g):
            for b in range(2):
                iteration(g + b, b)

        iteration(n_chunks - 2, 0, fire_idx2=False)
        iteration(n_chunks - 1, 1, fire_next=False, fire_idx2=False)
        drain_out(1)

    return kern(tok2, emb_i, emb_v, emb_f)


def kernel(tokens, emb_i, emb_v, emb_f):
    lead = tokens.shape[:-1]
    B = tokens.shape[0] * tokens.shape[1]
    out = _three_hot_sc(tokens.reshape(B, 3), emb_i, emb_v, emb_f)
    return out.reshape(lead + (EMB,))


# speedup vs baseline: 2.6129x; 2.6129x over previous
import functools
import math

import jax
import jax.numpy as jnp
from jax import lax
from jax.experimental import pallas as pl
from jax.experimental.pallas import tpu as pltpu
from jax.experimental.pallas import tpu_sc as plsc

EMB = 64
LANES = 16
GRP = 128
SCALE = math.sqrt(EMB) / 3.0


def _deinterleave_tc(tokens):
    A, T, _ = tokens.shape
    blk = 128
    rows_per_blk = blk * T // GRP
    grid = A // blk

    def body(t_ref, oi_ref, ov_ref, of_ref):
        x = t_ref[...]
        for f, o_ref in enumerate((oi_ref, ov_ref, of_ref)):
            o_ref[...] = x[:, :, f].reshape(rows_per_blk, GRP)

    out = jax.ShapeDtypeStruct((A * T // GRP, GRP), jnp.int32)
    return pl.pallas_call(
        body,
        grid=(grid,),
        in_specs=[pl.BlockSpec((blk, T, 3), lambda i: (i, 0, 0))],
        out_specs=[pl.BlockSpec((rows_per_blk, GRP), lambda i: (i, 0))] * 3,
        out_shape=[out, out, out],
    )(tokens)


@functools.partial(jax.jit, static_argnames=("num_cores", "num_subcores", "chunk"))
def _three_hot_sc(idx_i, idx_v, idx_f, emb_i, emb_v, emb_f,
                  num_cores=2, num_subcores=16, chunk=256):
    n_rows, grp = idx_i.shape
    B = n_rows * GRP
    NW = num_cores * num_subcores
    per_w = B // NW
    groups = chunk // GRP
    n_chunks = per_w // chunk
    rows_per_w = per_w // GRP
    assert n_chunks % 2 == 0 and n_chunks >= 6

    mesh = plsc.VectorSubcoreMesh(core_axis_name="c", subcore_axis_name="s")

    idx_t = pltpu.VMEM((groups, GRP), jnp.int32)
    buf_t = pltpu.VMEM((chunk, EMB), jnp.float32)

    @functools.partial(
        pl.kernel,
        out_type=jax.ShapeDtypeStruct((B, EMB), jnp.float32),
        mesh=mesh,
        compiler_params=pltpu.CompilerParams(use_tc_tiling_on_sc=False),
        scratch_types=[
            [idx_t] * 3, [idx_t] * 3,
            [buf_t] * 3, [buf_t] * 3,
            [pltpu.SemaphoreType.DMA] * 2,
            [pltpu.SemaphoreType.DMA] * 2,
            pltpu.SemaphoreType.DMA,
        ],
    )
    def kern(ii, iv, iff, ti, tv, tf, out, x0, x1, b0, b1, gsem, osem, isem):
        wid = lax.axis_index("s") * num_cores + lax.axis_index("c")
        xs = (x0, x1)
        bs = (b0, b1)
        tabs = (ti, tv, tf)

        def row0_of(c):
            return wid * rows_per_w + c * groups

        def fire_idx(c, p):
            for t, (src, dst) in enumerate(zip((ii, iv, iff), xs[p])):
                pltpu.async_copy(src.at[pl.ds(row0_of(c), groups)], dst, isem)

        def wait_idx(p):
            for dst in xs[p]:
                pltpu.make_async_copy(
                    ii.at[pl.ds(0, groups)], dst, isem).wait()

        def fire_gathers(p):
            for t in range(3):
                for j in range(groups):
                    pltpu.async_copy(
                        tabs[t].at[xs[p][t].at[j]],
                        bs[p][t].at[pl.ds(j * GRP, GRP)], gsem[p])

        def drain_gathers(p):
            for t in range(3):
                for j in range(groups):
                    pltpu.make_async_copy(
                        tabs[t].at[pl.ds(0, GRP)],
                        bs[p][t].at[pl.ds(j * GRP, GRP)], gsem[p]).wait()

        def compute(p):
            bi, bv, bf = bs[p]

            def row_body(r, _):
                for q in range(EMB // LANES):
                    s = pl.ds(q * LANES, LANES)
                    bi[r, s] = (bi[r, s] + bv[r, s] + bf[r, s]) * SCALE
                return ()

            lax.fori_loop(0, chunk, row_body, ())

        def fire_out(c, p):
            pltpu.async_copy(
                bs[p][0], out.at[pl.ds(row0_of(c) * GRP, chunk)], osem[p])

        def drain_out(p):
            pltpu.make_async_copy(
                bs[p][0], out.at[pl.ds(0, chunk)], osem[p]).wait()

        def iteration(c, p, *, first=False, fire_next=True, fire_idx2=True):
            q = p ^ 1
            if not first:
                drain_out(q)
            if fire_next:
                wait_idx(q)
                fire_gathers(q)
            drain_gathers(p)
            if fire_idx2:
                fire_idx(c + 2, p)
            compute(p)
            fire_out(c, p)

        for src, dst in zip((ii, iv, iff), xs[0]):
            pltpu.sync_copy(src.at[pl.ds(row0_of(0), groups)], dst)
        fire_gathers(0)
        fire_idx(1, 1)

        iteration(0, 0, first=True)
        iteration(1, 1)

        @pl.loop(2, n_chunks - 2, step=2)
        def steady(g):
            for b in range(2):
                iteration(g + b, b)

        iteration(n_chunks - 2, 0, fire_idx2=False)
        iteration(n_chunks - 1, 1, fire_next=False, fire_idx2=False)
        drain_out(1)

    return kern(idx_i, idx_v, idx_f, emb_i, emb_v, emb_f)


def kernel(tokens, emb_i, emb_v, emb_f):
    lead = tokens.shape[:-1]
    B = tokens.shape[0] * tokens.shape[1]
    idx_i, idx_v, idx_f = _deinterleave_tc(tokens)
    out = _three_hot_sc(idx_i, idx_v, idx_f, emb_i, emb_v, emb_f)
    return out.reshape(lead + (EMB,))


# speedup vs baseline: 3.9293x; 1.5038x over previous
import functools
import math

import jax
import jax.numpy as jnp
from jax import lax
from jax.experimental import pallas as pl
from jax.experimental.pallas import tpu as pltpu
from jax.experimental.pallas import tpu_sc as plsc

EMB = 64
LANES = 16
GRP = 128
SCALE = math.sqrt(EMB) / 3.0


def _deinterleave_tc(tokens):
    A, T, _ = tokens.shape
    blk = 128
    rows_per_blk = blk * T // GRP
    grid = A // blk

    def body(t_ref, oi_ref, ov_ref, of_ref):
        x = t_ref[...]
        for f, o_ref in enumerate((oi_ref, ov_ref, of_ref)):
            o_ref[...] = x[:, :, f].reshape(rows_per_blk, GRP)

    out = jax.ShapeDtypeStruct((A * T // GRP, GRP), jnp.int32)
    return pl.pallas_call(
        body,
        grid=(grid,),
        in_specs=[pl.BlockSpec((blk, T, 3), lambda i: (i, 0, 0))],
        out_specs=[pl.BlockSpec((rows_per_blk, GRP), lambda i: (i, 0))] * 3,
        out_shape=[out, out, out],
    )(tokens)


@functools.partial(jax.jit, static_argnames=("num_cores", "num_subcores", "chunk"))
def _three_hot_sc(idx_i, idx_v, idx_f, emb_i, emb_v, emb_f,
                  num_cores=2, num_subcores=16, chunk=256):
    n_rows, grp = idx_i.shape
    B = n_rows * GRP
    NW = num_cores * num_subcores
    per_w = B // NW
    groups = chunk // GRP
    n_chunks = per_w // chunk
    rows_per_w = per_w // GRP
    assert n_chunks % 2 == 0 and n_chunks >= 6

    mesh = plsc.VectorSubcoreMesh(core_axis_name="c", subcore_axis_name="s")

    idx_t = pltpu.VMEM((groups, GRP), jnp.int32)
    buf_t = pltpu.VMEM((chunk, EMB), jnp.float32)

    @functools.partial(
        pl.kernel,
        out_type=jax.ShapeDtypeStruct((B, EMB), jnp.float32),
        mesh=mesh,
        compiler_params=pltpu.CompilerParams(use_tc_tiling_on_sc=False),
        scratch_types=[
            [idx_t] * 3, [idx_t] * 3,
            [buf_t] * 3, [buf_t] * 3,
            [pltpu.SemaphoreType.DMA] * 2,
            [pltpu.SemaphoreType.DMA] * 2,
            pltpu.SemaphoreType.DMA,
        ],
    )
    def kern(ii, iv, iff, ti, tv, tf, out, x0, x1, b0, b1, gsem, osem, isem):
        wid = lax.axis_index("s") * num_cores + lax.axis_index("c")
        xs = (x0, x1)
        bs = (b0, b1)
        tabs = (ti, tv, tf)

        def row0_of(c):
            return wid * rows_per_w + c * groups

        def fire_idx(c, p):
            for t, (src, dst) in enumerate(zip((ii, iv, iff), xs[p])):
                pltpu.async_copy(src.at[pl.ds(row0_of(c), groups)], dst, isem)

        def wait_idx(p):
            for dst in xs[p]:
                pltpu.make_async_copy(
                    ii.at[pl.ds(0, groups)], dst, isem).wait()

        def fire_gathers(p):
            for t in range(3):
                for j in range(groups):
                    pltpu.async_copy(
                        tabs[t].at[xs[p][t].at[j]],
                        bs[p][t].at[pl.ds(j * GRP, GRP)], gsem[p])

        def drain_gathers(p):
            for t in range(3):
                for j in range(groups):
                    pltpu.make_async_copy(
                        tabs[t].at[pl.ds(0, GRP)],
                        bs[p][t].at[pl.ds(j * GRP, GRP)], gsem[p]).wait()

        def compute(p):
            bi, bv, bf = bs[p]

            def row_body(r, _):
                for q in range(EMB // LANES):
                    s = pl.ds(q * LANES, LANES)
                    bi[r, s] = (bi[r, s] + bv[r, s] + bf[r, s]) * SCALE
                return ()

            lax.fori_loop(0, chunk, row_body, ())

        def fire_out(c, p):
            pltpu.async_copy(
                bs[p][0], out.at[pl.ds(row0_of(c) * GRP, chunk)], osem[p])

        def drain_out(p):
            pltpu.make_async_copy(
                bs[p][0], out.at[pl.ds(0, chunk)], osem[p]).wait()

        def iteration(c, p, *, first=False, fire_next=True, fire_idx2=True):
            q = p ^ 1
            if not first:
                drain_out(q)
            if fire_next:
                wait_idx(q)
                fire_gathers(q)
            drain_gathers(p)
            if fire_idx2:
                fire_idx(c + 2, p)
            compute(p)
            fire_out(c, p)

        for src, dst in zip((ii, iv, iff), xs[0]):
            pltpu.sync_copy(src.at[pl.ds(row0_of(0), groups)], dst)
        fire_gathers(0)
        fire_idx(1, 1)

        iteration(0, 0, first=True)
        iteration(1, 1)

        @pl.loop(2, n_chunks - 2, step=2)
        def steady(g):
            for b in range(2):
                iteration(g + b, b)

        iteration(n_chunks - 2, 0, fire_idx2=False)
        iteration(n_chunks - 1, 1, fire_next=False, fire_idx2=False)
        drain_out(1)

    return kern(idx_i, idx_v, idx_f, emb_i, emb_v, emb_f)


def kernel(tokens, emb_i, emb_v, emb_f):
    lead = tokens.shape[:-1]
    B = tokens.shape[0] * tokens.shape[1]
    idx_i = tokens[:, :, 0].reshape(B // GRP, GRP)
    idx_v = tokens[:, :, 1].reshape(B // GRP, GRP)
    idx_f = tokens[:, :, 2].reshape(B // GRP, GRP)
    out = _three_hot_sc(idx_i, idx_v, idx_f, emb_i, emb_v, emb_f)
    return out.reshape(lead + (EMB,))


# speedup vs baseline: 3.9701x; 1.0104x over previous
import functools
import math

import jax
import jax.numpy as jnp
from jax import lax
from jax.experimental import pallas as pl
from jax.experimental.pallas import tpu as pltpu
from jax.experimental.pallas import tpu_sc as plsc

EMB = 64
LANES = 16
GRP = 128
SCALE = math.sqrt(EMB) / 3.0


def _deinterleave_tc(tokens):
    A, T, _ = tokens.shape
    blk = 128
    rows_per_blk = blk * T // GRP
    grid = A // blk

    def body(t_ref, oi_ref, ov_ref, of_ref):
        x = t_ref[...]
        for f, o_ref in enumerate((oi_ref, ov_ref, of_ref)):
            o_ref[...] = x[:, :, f].reshape(rows_per_blk, GRP)

    out = jax.ShapeDtypeStruct((A * T // GRP, GRP), jnp.int32)
    return pl.pallas_call(
        body,
        grid=(grid,),
        in_specs=[pl.BlockSpec((blk, T, 3), lambda i: (i, 0, 0))],
        out_specs=[pl.BlockSpec((rows_per_blk, GRP), lambda i: (i, 0))] * 3,
        out_shape=[out, out, out],
    )(tokens)


@functools.partial(jax.jit, static_argnames=("num_cores", "num_subcores", "chunk"))
def _three_hot_sc(idx3, emb_i, emb_v, emb_f,
                  num_cores=2, num_subcores=16, chunk=256):
    _, n_rows, grp = idx3.shape
    B = n_rows * GRP
    NW = num_cores * num_subcores
    per_w = B // NW
    groups = chunk // GRP
    n_chunks = per_w // chunk
    rows_per_w = per_w // GRP
    assert n_chunks % 2 == 0 and n_chunks >= 6

    mesh = plsc.VectorSubcoreMesh(core_axis_name="c", subcore_axis_name="s")

    idx_t = pltpu.VMEM((groups, GRP), jnp.int32)
    buf_t = pltpu.VMEM((chunk, EMB), jnp.float32)

    @functools.partial(
        pl.kernel,
        out_type=jax.ShapeDtypeStruct((B, EMB), jnp.float32),
        mesh=mesh,
        compiler_params=pltpu.CompilerParams(use_tc_tiling_on_sc=False),
        scratch_types=[
            [idx_t] * 3, [idx_t] * 3,
            [buf_t] * 3, [buf_t] * 3,
            [pltpu.SemaphoreType.DMA] * 2,
            [pltpu.SemaphoreType.DMA] * 2,
            pltpu.SemaphoreType.DMA,
        ],
    )
    def kern(ixs, ti, tv, tf, out, x0, x1, b0, b1, gsem, osem, isem):
        wid = lax.axis_index("s") * num_cores + lax.axis_index("c")
        xs = (x0, x1)
        bs = (b0, b1)
        tabs = (ti, tv, tf)

        def row0_of(c):
            return wid * rows_per_w + c * groups

        def fire_idx(c, p):
            for t, dst in enumerate(xs[p]):
                pltpu.async_copy(
                    ixs.at[t, pl.ds(row0_of(c), groups)], dst, isem)

        def wait_idx(p):
            for dst in xs[p]:
                pltpu.make_async_copy(
                    ixs.at[0, pl.ds(0, groups)], dst, isem).wait()

        def fire_gathers(p):
            for t in range(3):
                for j in range(groups):
                    pltpu.async_copy(
                        tabs[t].at[xs[p][t].at[j]],
                        bs[p][t].at[pl.ds(j * GRP, GRP)], gsem[p])

        def drain_gathers(p):
            for t in range(3):
                for j in range(groups):
                    pltpu.make_async_copy(
                        tabs[t].at[pl.ds(0, GRP)],
                        bs[p][t].at[pl.ds(j * GRP, GRP)], gsem[p]).wait()

        def compute(p):
            bi, bv, bf = bs[p]

            def row_body(r, _):
                for q in range(EMB // LANES):
                    s = pl.ds(q * LANES, LANES)
                    bi[r, s] = (bi[r, s] + bv[r, s] + bf[r, s]) * SCALE
                return ()

            lax.fori_loop(0, chunk, row_body, ())

        def fire_out(c, p):
            pltpu.async_copy(
                bs[p][0], out.at[pl.ds(row0_of(c) * GRP, chunk)], osem[p])

        def drain_out(p):
            pltpu.make_async_copy(
                bs[p][0], out.at[pl.ds(0, chunk)], osem[p]).wait()

        def iteration(c, p, *, first=False, fire_next=True, fire_idx2=True):
            q = p ^ 1
            if not first:
                drain_out(q)
            if fire_next:
                wait_idx(q)
                fire_gathers(q)
            drain_gathers(p)
            if fire_idx2:
                fire_idx(c + 2, p)
            compute(p)
            fire_out(c, p)

        for t, dst in enumerate(xs[0]):
            pltpu.sync_copy(ixs.at[t, pl.ds(row0_of(0), groups)], dst)
        fire_gathers(0)
        fire_idx(1, 1)

        iteration(0, 0, first=True)
        iteration(1, 1)

        @pl.loop(2, n_chunks - 2, step=2)
        def steady(g):
            for b in range(2):
                iteration(g + b, b)

        iteration(n_chunks - 2, 0, fire_idx2=False)
        iteration(n_chunks - 1, 1, fire_next=False, fire_idx2=False)
        drain_out(1)

    return kern(idx3, emb_i, emb_v, emb_f)


def kernel(tokens, emb_i, emb_v, emb_f):
    lead = tokens.shape[:-1]
    B = tokens.shape[0] * tokens.shape[1]
    idx3 = jnp.moveaxis(tokens, -1, 0).reshape(3, B // GRP, GRP)
    out = _three_hot_sc(idx3, emb_i, emb_v, emb_f)
    return out.reshape(lead + (EMB,))
